# baseline (speedup 1.0000x reference)
import functools

import jax
import jax.numpy as jnp
from jax import lax
from jax.experimental import pallas as pl
from jax.experimental.pallas import tpu as pltpu
from jax.experimental.pallas import tpu_sc as plsc

_K = 64
_N = 32768
_ROWS = 128
_NC = 2
_NS = 16
_NW = _NC * _NS
_RPW = _ROWS // _NW
_NBV = _N // 256
_NSV = _NBV // 16
_SVCAP = 2048
_CCAP = 4096


def _keyify(v):
    u = lax.bitcast_convert_type(v, jnp.uint32)
    return u ^ ((u >> jnp.uint32(31)) * jnp.uint32(0x7FFFFFFF)
                + jnp.uint32(0x80000000))


def _unkey(key):
    pos = key >> jnp.uint32(31)
    u = key ^ (jnp.uint32(0x80000000)
               + (jnp.uint32(1) - pos) * jnp.uint32(0x7FFFFFFF))
    return lax.bitcast_convert_type(u, jnp.float32)


def _popcnt(m):
    return plsc.all_reduce_population_count(m)


def _count(m):
    return _popcnt(m)[0]


def _select_kth_key(key_ref, nv, k):
    k_splat = jnp.full((16,), k, jnp.int32)

    def bit_step(i, t):
        sh = (jnp.uint32(31) - i.astype(jnp.uint32))
        cand = t | jnp.full((16,), jnp.uint32(1) << sh, jnp.uint32)

        @plsc.parallel_loop(0, nv, unroll=4, carry=jnp.zeros((16,), jnp.int32))
        def cnt(j, acc):
            kv = key_ref[pl.ds(j * 16, 16)]
            return acc + _popcnt(kv >= cand)

        return jnp.where(cnt >= k_splat, cand, t)

    return lax.fori_loop(0, 32, bit_step, jnp.zeros((16,), jnp.uint32))


def _body(x_hbm, o_hbm, rowbuf, rowbuf2, zbuf, bmax, skey,
          sv_id, sv_key, s2_id, c_idx, c_key, eq_idx, st_val, st_idx,
          st_val2, st_idx2, isem, osem):
    wid = lax.axis_index("s") * _NC + lax.axis_index("c")
    iota = jnp.arange(16, dtype=jnp.int32)
    zero16f = jnp.zeros((16,), jnp.float32)

    row0 = wid * _RPW
    bufs = [rowbuf, rowbuf2]
    ih = pltpu.async_copy(x_hbm.at[row0], bufs[0], isem)

    @plsc.parallel_loop(0, _N // 16, unroll=8)
    def _(i):
        zbuf[pl.ds(i * 16, 16)] = zero16f

    def select_row(row, rbuf, stv, sti):
        @plsc.parallel_loop(0, _NBV, unroll=8)
        def _(g):
            base = g * 256
            m = rbuf[pl.ds(base, 16)]
            for j in range(1, 16):
                m = jnp.maximum(m, rbuf[pl.ds(base + 16 * j, 16)])
            bmax[pl.ds(g * 16, 16)] = m

        @plsc.parallel_loop(0, _NSV, unroll=2)
        def _(h):
            base = h * 256
            m = bmax[pl.ds(base, 16)]
            for j in range(1, 16):
                m = jnp.maximum(m, bmax[pl.ds(base + 16 * j, 16)])
            skey[pl.ds(h * 16, 16)] = _keyify(m)

        tA = _select_kth_key(skey, _NSV, _K)

        def sbody(gg, ptr):
            vs, ms, cs = [], [], []
            for u in range(8):
                kv = _keyify(bmax[pl.ds((gg * 8 + u) * 16, 16)])
                m = kv >= tA
                vs.append(kv)
                ms.append(m)
                cs.append(_count(m))
            for u in range(8):
                plsc.store_compressed(sv_key.at[pl.ds(ptr, 16)], vs[u],
                                      mask=ms[u])
                plsc.store_compressed(sv_id.at[pl.ds(ptr, 16)],
                                      (gg * 8 + u) * 16 + iota, mask=ms[u])
                ptr = ptr + cs[u]
            return ptr

        n1 = lax.fori_loop(0, _NBV // 8, sbody, jnp.int32(0))
        nv1 = (n1 + 15) // 16
        sv_key[pl.ds(n1, 16)] = jnp.zeros((16,), jnp.uint32)

        t1 = _select_kth_key(sv_key, nv1, _K)

        def s2body(j, ptr):
            kv = sv_key[pl.ds(j * 16, 16)]
            ids = sv_id[pl.ds(j * 16, 16)]
            m = (kv >= t1) & ((j * 16 + iota) < n1)
            plsc.store_compressed(s2_id.at[pl.ds(ptr, 16)], ids, mask=m)
            return ptr + _count(m)

        n2 = lax.fori_loop(0, nv1, s2body, jnp.int32(0))
        s2_id[pl.ds(n2, 16)] = jnp.zeros((16,), jnp.int32)
        nb2 = (n2 + 15) // 16

        def cbody(j, ptr):
            ids = s2_id[pl.ds(j * 16, 16)]
            valid = (j * 16 + iota) < n2
            base = (ids >> 4) * 256 + (ids & 15)
            gv, gi, ms, cs = [], [], [], []
            for jj in range(16):
                idxv = base + 16 * jj
                kv = _keyify(plsc.load_gather(rbuf, [idxv]))
                m = (kv >= t1) & valid
                gv.append(kv)
                gi.append(idxv)
                ms.append(m)
                cs.append(_count(m))
            for jj in range(16):
                plsc.store_compressed(c_key.at[pl.ds(ptr, 16)], gv[jj],
                                      mask=ms[jj])
                plsc.store_compressed(c_idx.at[pl.ds(ptr, 16)], gi[jj],
                                      mask=ms[jj])
                ptr = jnp.minimum(ptr + cs[jj], _CCAP)
            return ptr

        nc = lax.fori_loop(0, nb2, cbody, jnp.int32(0))
        nvc = (nc + 15) // 16
        c_key[pl.ds(nc, 16)] = jnp.zeros((16,), jnp.uint32)

        tkey = _select_kth_key(c_key, nvc, _K)

        def cgbody(j, cnt):
            return cnt + _popcnt(c_key[pl.ds(j * 16, 16)] > tkey)

        cgv = lax.fori_loop(0, nvc, cgbody, jnp.zeros((16,), jnp.int32))
        need = jnp.full((16,), _K, jnp.int32) - cgv

        def eqbody(j, ptr):
            m = c_key[pl.ds(j * 16, 16)] == tkey
            plsc.store_compressed(eq_idx.at[pl.ds(ptr, 16)],
                                  c_idx[pl.ds(j * 16, 16)], mask=m)
            return ptr + _count(m)

        ne = lax.fori_loop(0, nvc, eqbody, jnp.int32(0))
        eq_idx[pl.ds(ne, 16)] = jnp.full((16,), jnp.int32(0x7FFFFFFF))
        nve = (ne + 15) // 16

        def lstep(i, L):
            cand = L | jnp.full((16,), jnp.int32(1) << (14 - i), jnp.int32)

            def lcnt(j, cnt):
                return cnt + _popcnt(eq_idx[pl.ds(j * 16, 16)] < cand)

            cnt = lax.fori_loop(0, nve, lcnt, jnp.zeros((16,), jnp.int32))
            return jnp.where(cnt <= need - 1, cand, L)

        L = lax.fori_loop(0, 15, lstep, jnp.zeros((16,), jnp.int32))

        def keepbody(j, ptr):
            kv = c_key[pl.ds(j * 16, 16)]
            iv = c_idx[pl.ds(j * 16, 16)]
            m = (kv > tkey) | ((kv == tkey) & (iv <= L))
            plsc.store_compressed(stv.at[pl.ds(ptr, 16)], _unkey(kv), mask=m)
            plsc.store_compressed(sti.at[pl.ds(ptr, 16)], iv, mask=m)
            return ptr + _count(m)

        lax.fori_loop(0, nvc, keepbody, jnp.int32(0))

    sts = [(st_val, st_idx), (st_val2, st_idx2)]
    oh = None
    for r in range(_RPW):
        ih.wait()
        if r + 1 < _RPW:
            ih = pltpu.async_copy(x_hbm.at[row0 + r + 1], bufs[(r + 1) % 2],
                                  isem)
        stv, sti = sts[r % 2]
        select_row(row0 + r, bufs[r % 2], stv, sti)
        if oh is not None:
            oh.wait()
            _, psti = sts[(r + 1) % 2]
            for j in range(_K // 16):
                plsc.store_scatter(zbuf, [psti[pl.ds(j * 16, 16)]], zero16f)
        for j in range(_K // 16):
            plsc.store_scatter(zbuf, [sti[pl.ds(j * 16, 16)]],
                               stv[pl.ds(j * 16, 16)])
        oh = pltpu.async_copy(zbuf, o_hbm.at[row0 + r], osem)
    oh.wait()


_sc_call = functools.partial(
    pl.kernel,
    mesh=plsc.VectorSubcoreMesh(core_axis_name="c", subcore_axis_name="s"),
    compiler_params=pltpu.CompilerParams(needs_layout_passes=False),
    out_type=jax.ShapeDtypeStruct((_ROWS, _N), jnp.float32),
    scratch_types=[
        pltpu.VMEM((_N,), jnp.float32),
        pltpu.VMEM((_N,), jnp.float32),
        pltpu.VMEM((_N,), jnp.float32),
        pltpu.VMEM((_N // 16,), jnp.float32),
        pltpu.VMEM((_NBV,), jnp.uint32),
        pltpu.VMEM((_SVCAP + 16,), jnp.int32),
        pltpu.VMEM((_SVCAP + 16,), jnp.uint32),
        pltpu.VMEM((_SVCAP + 16,), jnp.int32),
        pltpu.VMEM((_CCAP + 16,), jnp.int32),
        pltpu.VMEM((_CCAP + 16,), jnp.uint32),
        pltpu.VMEM((_CCAP + 16,), jnp.int32),
        pltpu.VMEM((_K + 16,), jnp.float32),
        pltpu.VMEM((_K + 16,), jnp.int32),
        pltpu.VMEM((_K + 16,), jnp.float32),
        pltpu.VMEM((_K + 16,), jnp.int32),
        pltpu.SemaphoreType.DMA,
        pltpu.SemaphoreType.DMA,
    ],
)(_body)


def kernel(x):
    return _sc_call(x)


# speedup vs baseline: 1.0322x; 1.0322x over previous
import functools

import jax
import jax.numpy as jnp
from jax import lax
from jax.experimental import pallas as pl
from jax.experimental.pallas import tpu as pltpu
from jax.experimental.pallas import tpu_sc as plsc

_K = 64
_N = 32768
_ROWS = 128
_NC = 2
_NS = 16
_NW = _NC * _NS
_RPW = _ROWS // _NW
_NBV = _N // 256
_NSV = _NBV // 16
_SVCAP = 2048
_CCAP = 4096


def _keyify(v):
    u = lax.bitcast_convert_type(v, jnp.uint32)
    return u ^ ((u >> jnp.uint32(31)) * jnp.uint32(0x7FFFFFFF)
                + jnp.uint32(0x80000000))


def _unkey(key):
    pos = key >> jnp.uint32(31)
    u = key ^ (jnp.uint32(0x80000000)
               + (jnp.uint32(1) - pos) * jnp.uint32(0x7FFFFFFF))
    return lax.bitcast_convert_type(u, jnp.float32)


def _popcnt(m):
    return plsc.all_reduce_population_count(m)


def _count(m):
    return _popcnt(m)[0]


def _select_kth_key(key_ref, nv, k):
    k_splat = jnp.full((16,), k, jnp.int32)

    def bit_step(i, t):
        sh = (jnp.uint32(31) - i.astype(jnp.uint32))
        cand = t | jnp.full((16,), jnp.uint32(1) << sh, jnp.uint32)

        @plsc.parallel_loop(0, nv, unroll=4, carry=jnp.zeros((16,), jnp.int32))
        def cnt(j, acc):
            kv = key_ref[pl.ds(j * 16, 16)]
            return acc + _popcnt(kv >= cand)

        return jnp.where(cnt >= k_splat, cand, t)

    return lax.fori_loop(0, 32, bit_step, jnp.zeros((16,), jnp.uint32))


def _body(x_hbm, o_hbm, rowbuf, rowbuf2, zbuf, bmax, skey,
          sv_id, sv_key, s2_id, c_idx, c_key, eq_idx, st_val, st_idx,
          st_val2, st_idx2, isem, osem):
    wid = lax.axis_index("s") * _NC + lax.axis_index("c")
    iota = jnp.arange(16, dtype=jnp.int32)
    zero16f = jnp.zeros((16,), jnp.float32)

    row0 = wid * _RPW
    bufs = [rowbuf, rowbuf2]
    ih = pltpu.async_copy(x_hbm.at[row0], bufs[0], isem)

    @plsc.parallel_loop(0, _N // 16, unroll=8)
    def _(i):
        zbuf[pl.ds(i * 16, 16)] = zero16f

    def select_row(row, rbuf, stv, sti):
        @plsc.parallel_loop(0, _NBV, unroll=4)
        def _(g):
            base = g * 256
            m = rbuf[pl.ds(base, 16)]
            for j in range(1, 16):
                m = jnp.maximum(m, rbuf[pl.ds(base + 16 * j, 16)])
            bmax[pl.ds(g * 16, 16)] = m

        @plsc.parallel_loop(0, _NSV, unroll=2)
        def _(h):
            base = h * 256
            m = bmax[pl.ds(base, 16)]
            for j in range(1, 16):
                m = jnp.maximum(m, bmax[pl.ds(base + 16 * j, 16)])
            skey[pl.ds(h * 16, 16)] = _keyify(m)

        tA = _select_kth_key(skey, _NSV, _K)

        def sbody(gg, ptr):
            vs, ms, cs = [], [], []
            for u in range(8):
                kv = _keyify(bmax[pl.ds((gg * 8 + u) * 16, 16)])
                m = kv >= tA
                vs.append(kv)
                ms.append(m)
                cs.append(_count(m))
            for u in range(8):
                plsc.store_compressed(sv_key.at[pl.ds(ptr, 16)], vs[u],
                                      mask=ms[u])
                plsc.store_compressed(sv_id.at[pl.ds(ptr, 16)],
                                      (gg * 8 + u) * 16 + iota, mask=ms[u])
                ptr = ptr + cs[u]
            return ptr

        n1 = lax.fori_loop(0, _NBV // 8, sbody, jnp.int32(0))
        nv1 = (n1 + 15) // 16
        sv_key[pl.ds(n1, 16)] = jnp.zeros((16,), jnp.uint32)

        t1 = _select_kth_key(sv_key, nv1, _K)

        def s2body(j, ptr):
            kv = sv_key[pl.ds(j * 16, 16)]
            ids = sv_id[pl.ds(j * 16, 16)]
            m = (kv >= t1) & ((j * 16 + iota) < n1)
            plsc.store_compressed(s2_id.at[pl.ds(ptr, 16)], ids, mask=m)
            return ptr + _count(m)

        n2 = lax.fori_loop(0, nv1, s2body, jnp.int32(0))
        s2_id[pl.ds(n2, 16)] = jnp.zeros((16,), jnp.int32)
        nb2 = (n2 + 15) // 16

        def cbody(j, ptr):
            ids = s2_id[pl.ds(j * 16, 16)]
            valid = (j * 16 + iota) < n2
            base = (ids >> 4) * 256 + (ids & 15)
            gv, gi, ms, cs = [], [], [], []
            for jj in range(16):
                idxv = base + 16 * jj
                kv = _keyify(plsc.load_gather(rbuf, [idxv]))
                m = (kv >= t1) & valid
                gv.append(kv)
                gi.append(idxv)
                ms.append(m)
                cs.append(_count(m))
            for jj in range(16):
                plsc.store_compressed(c_key.at[pl.ds(ptr, 16)], gv[jj],
                                      mask=ms[jj])
                plsc.store_compressed(c_idx.at[pl.ds(ptr, 16)], gi[jj],
                                      mask=ms[jj])
                ptr = jnp.minimum(ptr + cs[jj], _CCAP)
            return ptr

        nc = lax.fori_loop(0, nb2, cbody, jnp.int32(0))
        nvc = (nc + 15) // 16
        c_key[pl.ds(nc, 16)] = jnp.zeros((16,), jnp.uint32)

        tkey = _select_kth_key(c_key, nvc, _K)

        def cgbody(j, cnt):
            return cnt + _popcnt(c_key[pl.ds(j * 16, 16)] > tkey)

        cgv = lax.fori_loop(0, nvc, cgbody, jnp.zeros((16,), jnp.int32))
        need = jnp.full((16,), _K, jnp.int32) - cgv

        def eqbody(j, ptr):
            m = c_key[pl.ds(j * 16, 16)] == tkey
            plsc.store_compressed(eq_idx.at[pl.ds(ptr, 16)],
                                  c_idx[pl.ds(j * 16, 16)], mask=m)
            return ptr + _count(m)

        ne = lax.fori_loop(0, nvc, eqbody, jnp.int32(0))
        eq_idx[pl.ds(ne, 16)] = jnp.full((16,), jnp.int32(0x7FFFFFFF))
        nve = (ne + 15) // 16

        def lstep(i, L):
            cand = L | jnp.full((16,), jnp.int32(1) << (14 - i), jnp.int32)

            def lcnt(j, cnt):
                return cnt + _popcnt(eq_idx[pl.ds(j * 16, 16)] < cand)

            cnt = lax.fori_loop(0, nve, lcnt, jnp.zeros((16,), jnp.int32))
            return jnp.where(cnt <= need - 1, cand, L)

        L = lax.fori_loop(0, 15, lstep, jnp.zeros((16,), jnp.int32))

        def keepbody(j, ptr):
            kv = c_key[pl.ds(j * 16, 16)]
            iv = c_idx[pl.ds(j * 16, 16)]
            m = (kv > tkey) | ((kv == tkey) & (iv <= L))
            plsc.store_compressed(stv.at[pl.ds(ptr, 16)], _unkey(kv), mask=m)
            plsc.store_compressed(sti.at[pl.ds(ptr, 16)], iv, mask=m)
            return ptr + _count(m)

        lax.fori_loop(0, nvc, keepbody, jnp.int32(0))

    sts = [(st_val, st_idx), (st_val2, st_idx2)]
    oh = None
    for r in range(_RPW):
        ih.wait()
        if r + 1 < _RPW:
            ih = pltpu.async_copy(x_hbm.at[row0 + r + 1], bufs[(r + 1) % 2],
                                  isem)
        stv, sti = sts[r % 2]
        select_row(row0 + r, bufs[r % 2], stv, sti)
        if oh is not None:
            oh.wait()
            _, psti = sts[(r + 1) % 2]
            for j in range(_K // 16):
                plsc.store_scatter(zbuf, [psti[pl.ds(j * 16, 16)]], zero16f)
        for j in range(_K // 16):
            plsc.store_scatter(zbuf, [sti[pl.ds(j * 16, 16)]],
                               stv[pl.ds(j * 16, 16)])
        oh = pltpu.async_copy(zbuf, o_hbm.at[row0 + r], osem)
    oh.wait()


_sc_call = functools.partial(
    pl.kernel,
    mesh=plsc.VectorSubcoreMesh(core_axis_name="c", subcore_axis_name="s"),
    compiler_params=pltpu.CompilerParams(needs_layout_passes=False),
    out_type=jax.ShapeDtypeStruct((_ROWS, _N), jnp.float32),
    scratch_types=[
        pltpu.VMEM((_N,), jnp.float32),
        pltpu.VMEM((_N,), jnp.float32),
        pltpu.VMEM((_N,), jnp.float32),
        pltpu.VMEM((_N // 16,), jnp.float32),
        pltpu.VMEM((_NBV,), jnp.uint32),
        pltpu.VMEM((_SVCAP + 16,), jnp.int32),
        pltpu.VMEM((_SVCAP + 16,), jnp.uint32),
        pltpu.VMEM((_SVCAP + 16,), jnp.int32),
        pltpu.VMEM((_CCAP + 16,), jnp.int32),
        pltpu.VMEM((_CCAP + 16,), jnp.uint32),
        pltpu.VMEM((_CCAP + 16,), jnp.int32),
        pltpu.VMEM((_K + 16,), jnp.float32),
        pltpu.VMEM((_K + 16,), jnp.int32),
        pltpu.VMEM((_K + 16,), jnp.float32),
        pltpu.VMEM((_K + 16,), jnp.int32),
        pltpu.SemaphoreType.DMA,
        pltpu.SemaphoreType.DMA,
    ],
)(_body)


def kernel(x):
    return _sc_call(x)


# speedup vs baseline: 1.0447x; 1.0121x over previous
import functools

import jax
import jax.numpy as jnp
from jax import lax
from jax.experimental import pallas as pl
from jax.experimental.pallas import tpu as pltpu
from jax.experimental.pallas import tpu_sc as plsc

_K = 64
_N = 32768
_ROWS = 128
_NC = 2
_NS = 16
_NW = _NC * _NS
_RPW = _ROWS // _NW
_NBV = _N // 256
_NSV = _NBV // 16
_SVCAP = 2048
_CCAP = 4096


def _keyify(v):
    u = lax.bitcast_convert_type(v, jnp.uint32)
    return u ^ ((u >> jnp.uint32(31)) * jnp.uint32(0x7FFFFFFF)
                + jnp.uint32(0x80000000))


def _unkey(key):
    pos = key >> jnp.uint32(31)
    u = key ^ (jnp.uint32(0x80000000)
               + (jnp.uint32(1) - pos) * jnp.uint32(0x7FFFFFFF))
    return lax.bitcast_convert_type(u, jnp.float32)


def _popcnt(m):
    return plsc.all_reduce_population_count(m)


def _count(m):
    return _popcnt(m)[0]


def _select_kth_key(key_ref, nv, k):
    k_splat = jnp.full((16,), k, jnp.int32)

    def bit_step(i, t):
        sh = (jnp.uint32(31) - i.astype(jnp.uint32))
        cand = t | jnp.full((16,), jnp.uint32(1) << sh, jnp.uint32)

        @plsc.parallel_loop(0, nv, unroll=4, carry=jnp.zeros((16,), jnp.int32))
        def cnt(j, acc):
            kv = key_ref[pl.ds(j * 16, 16)]
            return acc + _popcnt(kv >= cand)

        return jnp.where(cnt >= k_splat, cand, t)

    return lax.fori_loop(0, 32, bit_step, jnp.zeros((16,), jnp.uint32))


def _body(x_hbm, o_hbm, rowbuf, rowbuf2, zbuf, bmax, skey,
          sv_id, sv_key, s2_id, c_idx, c_key, eq_idx, st_val, st_idx,
          st_val2, st_idx2, isem, osem):
    wid = lax.axis_index("s") * _NC + lax.axis_index("c")
    iota = jnp.arange(16, dtype=jnp.int32)
    zero16f = jnp.zeros((16,), jnp.float32)

    row0 = wid * _RPW
    bufs = [rowbuf, rowbuf2]
    ih = pltpu.async_copy(x_hbm.at[row0], bufs[0], isem)

    @plsc.parallel_loop(0, _N // 16, unroll=8)
    def _(i):
        zbuf[pl.ds(i * 16, 16)] = zero16f

    def select_row(row, rbuf, stv, sti):
        @plsc.parallel_loop(0, _NBV, unroll=4)
        def _(g):
            base = g * 256
            m = rbuf[pl.ds(base, 16)]
            for j in range(1, 16):
                m = jnp.maximum(m, rbuf[pl.ds(base + 16 * j, 16)])
            bmax[pl.ds(g * 16, 16)] = m

        @plsc.parallel_loop(0, _NSV, unroll=2)
        def _(h):
            base = h * 256
            m = bmax[pl.ds(base, 16)]
            for j in range(1, 16):
                m = jnp.maximum(m, bmax[pl.ds(base + 16 * j, 16)])
            skey[pl.ds(h * 16, 16)] = _keyify(m)

        tA = _select_kth_key(skey, _NSV, _K)

        def sbody(gg, ptr):
            vs, ms, cs = [], [], []
            for u in range(8):
                kv = _keyify(bmax[pl.ds((gg * 8 + u) * 16, 16)])
                m = kv >= tA
                vs.append(kv)
                ms.append(m)
                cs.append(_count(m))
            for u in range(8):
                plsc.store_compressed(sv_key.at[pl.ds(ptr, 16)], vs[u],
                                      mask=ms[u])
                plsc.store_compressed(sv_id.at[pl.ds(ptr, 16)],
                                      (gg * 8 + u) * 16 + iota, mask=ms[u])
                ptr = ptr + cs[u]
            return ptr

        n1 = lax.fori_loop(0, _NBV // 8, sbody, jnp.int32(0))
        nv1 = (n1 + 15) // 16
        sv_key[pl.ds(n1, 16)] = jnp.zeros((16,), jnp.uint32)

        t1 = _select_kth_key(sv_key, nv1, _K)

        def s2body(j, ptr):
            kv = sv_key[pl.ds(j * 16, 16)]
            ids = sv_id[pl.ds(j * 16, 16)]
            m = (kv >= t1) & ((j * 16 + iota) < n1)
            plsc.store_compressed(s2_id.at[pl.ds(ptr, 16)], ids, mask=m)
            return ptr + _count(m)

        n2 = lax.fori_loop(0, nv1, s2body, jnp.int32(0))
        s2_id[pl.ds(n2, 16)] = jnp.zeros((16,), jnp.int32)
        nb2 = (n2 + 15) // 16

        def cbody(j, ptr):
            ids = s2_id[pl.ds(j * 16, 16)]
            valid = (j * 16 + iota) < n2
            base = (ids >> 4) * 256 + (ids & 15)
            gv, gi, ms, cs = [], [], [], []
            for jj in range(16):
                idxv = base + 16 * jj
                kv = _keyify(plsc.load_gather(rbuf, [idxv]))
                m = (kv >= t1) & valid
                gv.append(kv)
                gi.append(idxv)
                ms.append(m)
                cs.append(_count(m))
            for jj in range(16):
                plsc.store_compressed(c_key.at[pl.ds(ptr, 16)], gv[jj],
                                      mask=ms[jj])
                plsc.store_compressed(c_idx.at[pl.ds(ptr, 16)], gi[jj],
                                      mask=ms[jj])
                ptr = jnp.minimum(ptr + cs[jj], _CCAP)
            return ptr

        nc = lax.fori_loop(0, nb2, cbody, jnp.int32(0))
        nvc = (nc + 15) // 16
        c_key[pl.ds(nc, 16)] = jnp.zeros((16,), jnp.uint32)

        tkey = _select_kth_key(c_key, nvc, _K)

        def cgbody(j, cnt):
            return cnt + _popcnt(c_key[pl.ds(j * 16, 16)] > tkey)

        cgv = lax.fori_loop(0, nvc, cgbody, jnp.zeros((16,), jnp.int32))
        need = jnp.full((16,), _K, jnp.int32) - cgv

        def nebody(j, cnt):
            return cnt + _popcnt(c_key[pl.ds(j * 16, 16)] == tkey)

        nev = lax.fori_loop(0, nvc, nebody, jnp.zeros((16,), jnp.int32))

        def tie_easy():
            return jnp.full((16,), jnp.int32(0x7FFFFFFF))

        def tie_hard():
            def eqbody(j, ptr):
                m = c_key[pl.ds(j * 16, 16)] == tkey
                plsc.store_compressed(eq_idx.at[pl.ds(ptr, 16)],
                                      c_idx[pl.ds(j * 16, 16)], mask=m)
                return ptr + _count(m)

            ne = lax.fori_loop(0, nvc, eqbody, jnp.int32(0))
            eq_idx[pl.ds(ne, 16)] = jnp.full((16,), jnp.int32(0x7FFFFFFF))
            nve = (ne + 15) // 16

            def lstep(i, L):
                cand = L | jnp.full((16,), jnp.int32(1) << (14 - i), jnp.int32)

                def lcnt(j, cnt):
                    return cnt + _popcnt(eq_idx[pl.ds(j * 16, 16)] < cand)

                cnt = lax.fori_loop(0, nve, lcnt, jnp.zeros((16,), jnp.int32))
                return jnp.where(cnt <= need - 1, cand, L)

            return lax.fori_loop(0, 15, lstep, jnp.zeros((16,), jnp.int32))

        L = lax.cond((nev - need)[0] == 0, tie_easy, tie_hard)

        def keepbody(j, ptr):
            kv = c_key[pl.ds(j * 16, 16)]
            iv = c_idx[pl.ds(j * 16, 16)]
            m = (kv > tkey) | ((kv == tkey) & (iv <= L))
            plsc.store_compressed(stv.at[pl.ds(ptr, 16)], _unkey(kv), mask=m)
            plsc.store_compressed(sti.at[pl.ds(ptr, 16)], iv, mask=m)
            return ptr + _count(m)

        lax.fori_loop(0, nvc, keepbody, jnp.int32(0))

    sts = [(st_val, st_idx), (st_val2, st_idx2)]
    oh = None
    for r in range(_RPW):
        ih.wait()
        if r + 1 < _RPW:
            ih = pltpu.async_copy(x_hbm.at[row0 + r + 1], bufs[(r + 1) % 2],
                                  isem)
        stv, sti = sts[r % 2]
        select_row(row0 + r, bufs[r % 2], stv, sti)
        if oh is not None:
            oh.wait()
            _, psti = sts[(r + 1) % 2]
            for j in range(_K // 16):
                plsc.store_scatter(zbuf, [psti[pl.ds(j * 16, 16)]], zero16f)
        for j in range(_K // 16):
            plsc.store_scatter(zbuf, [sti[pl.ds(j * 16, 16)]],
                               stv[pl.ds(j * 16, 16)])
        oh = pltpu.async_copy(zbuf, o_hbm.at[row0 + r], osem)
    oh.wait()


_sc_call = functools.partial(
    pl.kernel,
    mesh=plsc.VectorSubcoreMesh(core_axis_name="c", subcore_axis_name="s"),
    compiler_params=pltpu.CompilerParams(needs_layout_passes=False),
    out_type=jax.ShapeDtypeStruct((_ROWS, _N), jnp.float32),
    scratch_types=[
        pltpu.VMEM((_N,), jnp.float32),
        pltpu.VMEM((_N,), jnp.float32),
        pltpu.VMEM((_N,), jnp.float32),
        pltpu.VMEM((_N // 16,), jnp.float32),
        pltpu.VMEM((_NBV,), jnp.uint32),
        pltpu.VMEM((_SVCAP + 16,), jnp.int32),
        pltpu.VMEM((_SVCAP + 16,), jnp.uint32),
        pltpu.VMEM((_SVCAP + 16,), jnp.int32),
        pltpu.VMEM((_CCAP + 16,), jnp.int32),
        pltpu.VMEM((_CCAP + 16,), jnp.uint32),
        pltpu.VMEM((_CCAP + 16,), jnp.int32),
        pltpu.VMEM((_K + 16,), jnp.float32),
        pltpu.VMEM((_K + 16,), jnp.int32),
        pltpu.VMEM((_K + 16,), jnp.float32),
        pltpu.VMEM((_K + 16,), jnp.int32),
        pltpu.SemaphoreType.DMA,
        pltpu.SemaphoreType.DMA,
    ],
)(_body)


def kernel(x):
    return _sc_call(x)
